# baseline (speedup 1.0000x reference)
import functools

import jax
import jax.numpy as jnp
from jax import lax
from jax.experimental import pallas as pl
from jax.experimental.pallas import tpu as pltpu
from jax.experimental.pallas import tpu_sc as plsc

EPS = 1e-08
BATCH = 16384
NC = 2
NS = 16
NW = NC * NS
B_W = BATCH // NW
CHUNK = 128
N_CHUNK = B_W // CHUNK
N_GROUP = B_W // 16


def _sc_gather_body(idx_hbm, wv_hbm, wa_hbm, wt_hbm, out_hbm,
                    idx_v, idx3, out_cols, sem):
    wid = lax.axis_index("s") * NC + lax.axis_index("c")
    base = wid * B_W

    pltpu.sync_copy(idx_hbm.at[pl.ds(base, B_W)], idx_v)

    for g in range(N_GROUP):
        seg = pl.ds(g * 16, 16)
        i3 = idx_v[seg] * 3
        idx3[pl.ds(0 * B_W + g * 16, 16)] = i3
        idx3[pl.ds(1 * B_W + g * 16, 16)] = i3 + 1
        idx3[pl.ds(2 * B_W + g * 16, 16)] = i3 + 2

    descs = []
    for c in range(N_CHUNK):
        rows = pl.ds(c * CHUNK, CHUNK)
        for j in range(3):
            idx_c = idx3.at[pl.ds(j * B_W + c * CHUNK, CHUNK)]
            descs.append(pltpu.async_copy(
                wv_hbm.at[idx_c], out_cols.at[pl.ds(j * B_W + c * CHUNK, CHUNK)], sem))
            descs.append(pltpu.async_copy(
                wt_hbm.at[idx_c], out_cols.at[pl.ds((4 + j) * B_W + c * CHUNK, CHUNK)], sem))
        descs.append(pltpu.async_copy(
            wa_hbm.at[idx_v.at[rows]], out_cols.at[pl.ds(3 * B_W + c * CHUNK, CHUNK)], sem))
    for d in descs:
        d.wait()

    for f in range(7):
        pltpu.sync_copy(out_cols.at[pl.ds(f * B_W, B_W)],
                        out_hbm.at[pl.ds(f * BATCH + base, B_W)])


def _sc_out_struct():
    return jax.ShapeDtypeStruct((7 * BATCH,), jnp.float32)


@functools.lru_cache(maxsize=1)
def _sc_gather():
    return functools.partial(
        pl.kernel,
        out_type=_sc_out_struct(),
        mesh=plsc.VectorSubcoreMesh(core_axis_name="c", subcore_axis_name="s"),
        scratch_types=[
            pltpu.VMEM((B_W,), jnp.int32),
            pltpu.VMEM((3 * B_W,), jnp.int32),
            pltpu.VMEM((7 * B_W,), jnp.float32),
            pltpu.SemaphoreType.DMA,
        ],
    )(_sc_gather_body)


def _tc_math_body(f_ref, o_ref):
    v0 = f_ref[0]
    v1 = f_ref[1]
    v2 = f_ref[2]
    a = f_ref[3]
    t0 = f_ref[4]
    t1 = f_ref[5]
    t2 = f_ref[6]

    norm = jnp.sqrt(v0 * v0 + v1 * v1 + v2 * v2)
    inv = 1.0 / (norm + EPS)
    u0 = v0 * inv
    u1 = v1 * inv
    u2 = v2 * inv

    s = jnp.sin(a)
    oc = 1.0 - jnp.cos(a)
    m = u0 * u0 + u1 * u1 + u2 * u2
    o_ref[0] = 1.0 + oc * (u0 * u0 - m)
    o_ref[1] = oc * u0 * u1 - s * u2
    o_ref[2] = oc * u0 * u2 + s * u1
    o_ref[3] = t0
    o_ref[4] = oc * u1 * u0 + s * u2
    o_ref[5] = 1.0 + oc * (u1 * u1 - m)
    o_ref[6] = oc * u1 * u2 - s * u0
    o_ref[7] = t1
    o_ref[8] = oc * u2 * u0 - s * u1
    o_ref[9] = oc * u2 * u1 + s * u0
    o_ref[10] = 1.0 + oc * (u2 * u2 - m)
    o_ref[11] = t2
    o_ref[12] = jnp.zeros_like(a)
    o_ref[13] = jnp.zeros_like(a)
    o_ref[14] = jnp.zeros_like(a)
    o_ref[15] = jnp.ones_like(a)


def kernel(image_idx, W_v, W_alpha, W_T):
    idx = image_idx.astype(jnp.int32)
    wv_flat = W_v.reshape(-1)
    wa_flat = W_alpha.reshape(-1)
    wt_flat = W_T.reshape(-1)
    fields = _sc_gather()(idx, wv_flat, wa_flat, wt_flat)
    fields3 = fields.reshape(7, 128, 128)
    out16 = pl.pallas_call(
        _tc_math_body,
        out_shape=jax.ShapeDtypeStruct((16, 128, 128), jnp.float32),
    )(fields3)
    return out16.reshape(16, BATCH).T.reshape(BATCH, 4, 4)


# speedup vs baseline: 34.5347x; 34.5347x over previous
import functools

import jax
import jax.numpy as jnp
from jax import lax
from jax.experimental import pallas as pl
from jax.experimental.pallas import tpu as pltpu
from jax.experimental.pallas import tpu_sc as plsc

EPS = 1e-08
BATCH = 16384
NC = 2
NS = 16
NW = NC * NS
B_W = BATCH // NW
CHUNK = 128
N_CHUNK = B_W // CHUNK


def _sc_gather_body(idx_hbm, f0, f1, f2, f3, f4, f5, f6, out_hbm,
                    idx_v, out_cols, sem):
    wid = lax.axis_index("s") * NC + lax.axis_index("c")
    base = wid * B_W

    pltpu.sync_copy(idx_hbm.at[pl.ds(base, B_W)], idx_v)

    tables = (f0, f1, f2, f3, f4, f5, f6)
    descs = []
    for c in range(N_CHUNK):
        idx_c = idx_v.at[pl.ds(c * CHUNK, CHUNK)]
        for f in range(7):
            descs.append(pltpu.async_copy(
                tables[f].at[idx_c],
                out_cols.at[pl.ds(f * B_W + c * CHUNK, CHUNK)], sem))
    for d in descs:
        d.wait()

    for f in range(7):
        pltpu.sync_copy(out_cols.at[pl.ds(f * B_W, B_W)],
                        out_hbm.at[pl.ds(f * BATCH + base, B_W)])


@functools.lru_cache(maxsize=1)
def _sc_gather():
    return functools.partial(
        pl.kernel,
        out_type=jax.ShapeDtypeStruct((7 * BATCH,), jnp.float32),
        mesh=plsc.VectorSubcoreMesh(core_axis_name="c", subcore_axis_name="s"),
        scratch_types=[
            pltpu.VMEM((B_W,), jnp.int32),
            pltpu.VMEM((7 * B_W,), jnp.float32),
            pltpu.SemaphoreType.DMA,
        ],
    )(_sc_gather_body)


def _tc_math_body(f_ref, o_ref):
    v0 = f_ref[0]
    v1 = f_ref[1]
    v2 = f_ref[2]
    a = f_ref[3]
    t0 = f_ref[4]
    t1 = f_ref[5]
    t2 = f_ref[6]

    norm = jnp.sqrt(v0 * v0 + v1 * v1 + v2 * v2)
    inv = 1.0 / (norm + EPS)
    u0 = v0 * inv
    u1 = v1 * inv
    u2 = v2 * inv

    s = jnp.sin(a)
    oc = 1.0 - jnp.cos(a)
    m = u0 * u0 + u1 * u1 + u2 * u2
    o_ref[0] = 1.0 + oc * (u0 * u0 - m)
    o_ref[1] = oc * u0 * u1 - s * u2
    o_ref[2] = oc * u0 * u2 + s * u1
    o_ref[3] = t0
    o_ref[4] = oc * u1 * u0 + s * u2
    o_ref[5] = 1.0 + oc * (u1 * u1 - m)
    o_ref[6] = oc * u1 * u2 - s * u0
    o_ref[7] = t1
    o_ref[8] = oc * u2 * u0 - s * u1
    o_ref[9] = oc * u2 * u1 + s * u0
    o_ref[10] = 1.0 + oc * (u2 * u2 - m)
    o_ref[11] = t2
    o_ref[12] = jnp.zeros_like(a)
    o_ref[13] = jnp.zeros_like(a)
    o_ref[14] = jnp.zeros_like(a)
    o_ref[15] = jnp.ones_like(a)


def kernel(image_idx, W_v, W_alpha, W_T):
    idx = image_idx.astype(jnp.int32)
    fields = _sc_gather()(
        idx,
        W_v[:, 0], W_v[:, 1], W_v[:, 2],
        W_alpha[:, 0],
        W_T[:, 0], W_T[:, 1], W_T[:, 2],
    )
    fields3 = fields.reshape(7, 128, 128)
    out16 = pl.pallas_call(
        _tc_math_body,
        out_shape=jax.ShapeDtypeStruct((16, 128, 128), jnp.float32),
    )(fields3)
    return out16.reshape(16, BATCH).T.reshape(BATCH, 4, 4)


# speedup vs baseline: 34.5602x; 1.0007x over previous
import functools

import jax
import jax.numpy as jnp
from jax import lax
from jax.experimental import pallas as pl
from jax.experimental.pallas import tpu as pltpu
from jax.experimental.pallas import tpu_sc as plsc

EPS = 1e-08
BATCH = 16384
NC = 2
NS = 16
NW = NC * NS
B_W = BATCH // NW
CHUNK = 128
N_CHUNK = B_W // CHUNK
N_GROUP = B_W // 16

_INV_PI = 0.31830988618379067
_PI_HI = 3.140625
_PI_LO = 9.67653589793e-4
_MAGIC = 0x5F3759DF


def _rsqrt16(x):
    i = lax.bitcast_convert_type(x, jnp.int32)
    i = jnp.full((16,), _MAGIC, jnp.int32) - lax.shift_right_arithmetic(i, 1)
    y = lax.bitcast_convert_type(i, jnp.float32)
    for _ in range(3):
        y = y * (1.5 - 0.5 * x * y * y)
    return y


def _sincos16(a):
    shift = jnp.where(a >= 0.0, 0.5, -0.5)
    n = (a * _INV_PI + shift).astype(jnp.int32)
    nf = n.astype(jnp.float32)
    r = (a - nf * _PI_HI) - nf * _PI_LO
    r2 = r * r
    s = r * (1.0 + r2 * (-1.0 / 6.0 + r2 * (1.0 / 120.0 + r2 * (-1.0 / 5040.0 + r2 * (1.0 / 362880.0)))))
    c = 1.0 + r2 * (-0.5 + r2 * (1.0 / 24.0 + r2 * (-1.0 / 720.0 + r2 * (1.0 / 40320.0 + r2 * (-1.0 / 3628800.0)))))
    sgn = jnp.where((n & 1) == 0, 1.0, -1.0)
    return s * sgn, c * sgn


def _sc_body(idx_hbm, f0, f1, f2, f3, f4, f5, f6, out_hbm,
             idx_v, cols, stage, sem):
    wid = lax.axis_index("s") * NC + lax.axis_index("c")
    base = wid * B_W

    pltpu.sync_copy(idx_hbm.at[pl.ds(base, B_W)], idx_v)

    tables = (f0, f1, f2, f3, f4, f5, f6)
    descs = []
    for c in range(N_CHUNK):
        idx_c = idx_v.at[pl.ds(c * CHUNK, CHUNK)]
        for f in range(7):
            descs.append(pltpu.async_copy(
                tables[f].at[idx_c],
                cols.at[pl.ds(f * B_W + c * CHUNK, CHUNK)], sem))
    for d in descs:
        d.wait()

    zero = jnp.zeros((16,), jnp.float32)
    one = jnp.full((16,), 1.0, jnp.float32)
    for g in range(N_GROUP):
        v0 = cols[pl.ds(0 * B_W + g * 16, 16)]
        v1 = cols[pl.ds(1 * B_W + g * 16, 16)]
        v2 = cols[pl.ds(2 * B_W + g * 16, 16)]
        a = cols[pl.ds(3 * B_W + g * 16, 16)]
        t0 = cols[pl.ds(4 * B_W + g * 16, 16)]
        t1 = cols[pl.ds(5 * B_W + g * 16, 16)]
        t2 = cols[pl.ds(6 * B_W + g * 16, 16)]

        n2 = v0 * v0 + v1 * v1 + v2 * v2
        y = _rsqrt16(n2)
        inv = 1.0 / (n2 * y + EPS)
        u0 = v0 * inv
        u1 = v1 * inv
        u2 = v2 * inv

        s, cc = _sincos16(a)
        oc = 1.0 - cc
        m = u0 * u0 + u1 * u1 + u2 * u2

        vals = (
            1.0 + oc * (u0 * u0 - m),
            oc * u0 * u1 - s * u2,
            oc * u0 * u2 + s * u1,
            t0,
            oc * u1 * u0 + s * u2,
            1.0 + oc * (u1 * u1 - m),
            oc * u1 * u2 - s * u0,
            t1,
            oc * u2 * u0 - s * u1,
            oc * u2 * u1 + s * u0,
            1.0 + oc * (u2 * u2 - m),
            t2,
            zero, zero, zero, one,
        )
        for f in range(16):
            stage[pl.ds(f * B_W + g * 16, 16)] = vals[f]

    for f in range(16):
        pltpu.sync_copy(stage.at[pl.ds(f * B_W, B_W)],
                        out_hbm.at[pl.ds(f * BATCH + base, B_W)])


@functools.lru_cache(maxsize=1)
def _sc_fused():
    return functools.partial(
        pl.kernel,
        out_type=jax.ShapeDtypeStruct((BATCH * 16,), jnp.float32),
        mesh=plsc.VectorSubcoreMesh(core_axis_name="c", subcore_axis_name="s"),
        scratch_types=[
            pltpu.VMEM((B_W,), jnp.int32),
            pltpu.VMEM((7 * B_W,), jnp.float32),
            pltpu.VMEM((16 * B_W,), jnp.float32),
            pltpu.SemaphoreType.DMA,
        ],
    )(_sc_body)


def kernel(image_idx, W_v, W_alpha, W_T):
    idx = image_idx.astype(jnp.int32)
    out = _sc_fused()(
        idx,
        W_v[:, 0], W_v[:, 1], W_v[:, 2],
        W_alpha[:, 0],
        W_T[:, 0], W_T[:, 1], W_T[:, 2],
    )
    return out.reshape(16, BATCH).T.reshape(BATCH, 4, 4)


# speedup vs baseline: 94.4986x; 2.7343x over previous
import functools

import jax
import jax.numpy as jnp
from jax import lax
from jax.experimental import pallas as pl
from jax.experimental.pallas import tpu as pltpu
from jax.experimental.pallas import tpu_sc as plsc

EPS = 1e-08
BATCH = 16384
NC = 2
NS = 16
NW = NC * NS
B_W = BATCH // NW
CHUNK = 128
N_CHUNK = B_W // CHUNK
N_GROUP = B_W // 16

N_TAB = 1000000
NPAD = 1000064

_INV_PI = 0.31830988618379067
_PI_HI = 3.140625
_PI_LO = 9.67653589793e-4
_MAGIC = 0x5F3759DF


def _rsqrt16(x):
    i = lax.bitcast_convert_type(x, jnp.int32)
    i = jnp.full((16,), _MAGIC, jnp.int32) - lax.shift_right_arithmetic(i, 1)
    y = lax.bitcast_convert_type(i, jnp.float32)
    for _ in range(3):
        y = y * (1.5 - 0.5 * x * y * y)
    return y


def _sincos16(a):
    shift = jnp.where(a >= 0.0, 0.5, -0.5)
    n = (a * _INV_PI + shift).astype(jnp.int32)
    nf = n.astype(jnp.float32)
    r = (a - nf * _PI_HI) - nf * _PI_LO
    r2 = r * r
    s = r * (1.0 + r2 * (-1.0 / 6.0 + r2 * (1.0 / 120.0 + r2 * (-1.0 / 5040.0 + r2 * (1.0 / 362880.0)))))
    c = 1.0 + r2 * (-0.5 + r2 * (1.0 / 24.0 + r2 * (-1.0 / 720.0 + r2 * (1.0 / 40320.0 + r2 * (-1.0 / 3628800.0)))))
    sgn = jnp.where((n & 1) == 0, 1.0, -1.0)
    return s * sgn, c * sgn


def _sc_body(idx_hbm, y_hbm, out_hbm, idx_v, off_v, cols, stage, sem):
    wid = lax.axis_index("s") * NC + lax.axis_index("c")
    base = wid * B_W

    pltpu.sync_copy(idx_hbm.at[pl.ds(base, B_W)], idx_v)

    for g in range(N_GROUP):
        x = idx_v[pl.ds(g * 16, 16)]
        b = lax.shift_left(lax.shift_right_logical(x, 7), 10) + (x & 127)
        for f in range(7):
            off_v[pl.ds(f * B_W + g * 16, 16)] = b + (f * CHUNK)

    descs = []
    for c in range(N_CHUNK):
        for f in range(7):
            off_c = off_v.at[pl.ds(f * B_W + c * CHUNK, CHUNK)]
            descs.append(pltpu.async_copy(
                y_hbm.at[off_c],
                cols.at[pl.ds(f * B_W + c * CHUNK, CHUNK)], sem))
    for d in descs:
        d.wait()

    zero = jnp.zeros((16,), jnp.float32)
    one = jnp.full((16,), 1.0, jnp.float32)
    for g in range(N_GROUP):
        v0 = cols[pl.ds(0 * B_W + g * 16, 16)]
        v1 = cols[pl.ds(1 * B_W + g * 16, 16)]
        v2 = cols[pl.ds(2 * B_W + g * 16, 16)]
        a = cols[pl.ds(3 * B_W + g * 16, 16)]
        t0 = cols[pl.ds(4 * B_W + g * 16, 16)]
        t1 = cols[pl.ds(5 * B_W + g * 16, 16)]
        t2 = cols[pl.ds(6 * B_W + g * 16, 16)]

        n2 = v0 * v0 + v1 * v1 + v2 * v2
        y = _rsqrt16(n2)
        inv = 1.0 / (n2 * y + EPS)
        u0 = v0 * inv
        u1 = v1 * inv
        u2 = v2 * inv

        s, cc = _sincos16(a)
        oc = 1.0 - cc
        m = u0 * u0 + u1 * u1 + u2 * u2

        vals = (
            1.0 + oc * (u0 * u0 - m),
            oc * u0 * u1 - s * u2,
            oc * u0 * u2 + s * u1,
            t0,
            oc * u1 * u0 + s * u2,
            1.0 + oc * (u1 * u1 - m),
            oc * u1 * u2 - s * u0,
            t1,
            oc * u2 * u0 - s * u1,
            oc * u2 * u1 + s * u0,
            1.0 + oc * (u2 * u2 - m),
            t2,
            zero, zero, zero, one,
        )
        for f in range(16):
            stage[pl.ds(f * B_W + g * 16, 16)] = vals[f]

    for f in range(16):
        pltpu.sync_copy(stage.at[pl.ds(f * B_W, B_W)],
                        out_hbm.at[pl.ds(f * BATCH + base, B_W)])


@functools.lru_cache(maxsize=1)
def _sc_fused():
    return functools.partial(
        pl.kernel,
        out_type=jax.ShapeDtypeStruct((BATCH * 16,), jnp.float32),
        mesh=plsc.VectorSubcoreMesh(core_axis_name="c", subcore_axis_name="s"),
        scratch_types=[
            pltpu.VMEM((B_W,), jnp.int32),
            pltpu.VMEM((7 * B_W,), jnp.int32),
            pltpu.VMEM((7 * B_W,), jnp.float32),
            pltpu.VMEM((16 * B_W,), jnp.float32),
            pltpu.SemaphoreType.DMA,
        ],
    )(_sc_body)


def kernel(image_idx, W_v, W_alpha, W_T):
    idx = image_idx.astype(jnp.int32)
    pad = ((0, NPAD - N_TAB), (0, 0))
    Wvp = jnp.pad(W_v, pad)
    Wap = jnp.pad(W_alpha, pad)
    WTp = jnp.pad(W_T, pad)
    X = jnp.concatenate([Wvp.T, Wap.T, WTp.T, Wap.T], axis=0)
    Y = X.reshape(8, NPAD // CHUNK, CHUNK).transpose(1, 0, 2).reshape(-1)
    out = _sc_fused()(idx, Y)
    return out.reshape(16, BATCH).T.reshape(BATCH, 4, 4)


# speedup vs baseline: 97.9023x; 1.0360x over previous
import functools

import jax
import jax.numpy as jnp
from jax import lax
from jax.experimental import pallas as pl
from jax.experimental.pallas import tpu as pltpu
from jax.experimental.pallas import tpu_sc as plsc

EPS = 1e-08
BATCH = 16384
NC = 2
NS = 16
NW = NC * NS
B_W = BATCH // NW
CHUNK = 128
N_CHUNK = B_W // CHUNK
N_GROUP = B_W // 16

N_TAB = 1000000
NPAD = 1000064

_INV_PI = 0.31830988618379067
_PI_HI = 3.140625
_PI_LO = 9.67653589793e-4
_MAGIC = 0x5F3759DF


def _rsqrt16(x):
    i = lax.bitcast_convert_type(x, jnp.int32)
    i = jnp.full((16,), _MAGIC, jnp.int32) - lax.shift_right_arithmetic(i, 1)
    y = lax.bitcast_convert_type(i, jnp.float32)
    for _ in range(3):
        y = y * (1.5 - 0.5 * x * y * y)
    return y


def _sincos16(a):
    shift = jnp.where(a >= 0.0, 0.5, -0.5)
    n = (a * _INV_PI + shift).astype(jnp.int32)
    nf = n.astype(jnp.float32)
    r = (a - nf * _PI_HI) - nf * _PI_LO
    r2 = r * r
    s = r * (1.0 + r2 * (-1.0 / 6.0 + r2 * (1.0 / 120.0 + r2 * (-1.0 / 5040.0 + r2 * (1.0 / 362880.0)))))
    c = 1.0 + r2 * (-0.5 + r2 * (1.0 / 24.0 + r2 * (-1.0 / 720.0 + r2 * (1.0 / 40320.0 + r2 * (-1.0 / 3628800.0)))))
    sgn = jnp.where((n & 1) == 0, 1.0, -1.0)
    return s * sgn, c * sgn


def _sc_body(idx_hbm, y_hbm, out_hbm, idx_v, off_v, cols, stage, sem):
    wid = lax.axis_index("s") * NC + lax.axis_index("c")
    base = wid * B_W

    pltpu.sync_copy(idx_hbm.at[pl.ds(base, B_W)], idx_v)

    for g in range(N_GROUP):
        x = idx_v[pl.ds(g * 16, 16)]
        b = lax.shift_left(lax.shift_right_logical(x, 7), 10) + (x & 127)
        for f in range(7):
            off_v[pl.ds(f * B_W + g * 16, 16)] = b + (f * CHUNK)

    descs = []
    for c in range(N_CHUNK):
        for f in range(7):
            off_c = off_v.at[pl.ds(f * B_W + c * CHUNK, CHUNK)]
            descs.append(pltpu.async_copy(
                y_hbm.at[off_c],
                cols.at[pl.ds(f * B_W + c * CHUNK, CHUNK)], sem))
    for d in descs:
        d.wait()

    zero = jnp.zeros((16,), jnp.float32)
    one = jnp.full((16,), 1.0, jnp.float32)
    for g in range(N_GROUP):
        v0 = cols[pl.ds(0 * B_W + g * 16, 16)]
        v1 = cols[pl.ds(1 * B_W + g * 16, 16)]
        v2 = cols[pl.ds(2 * B_W + g * 16, 16)]
        a = cols[pl.ds(3 * B_W + g * 16, 16)]
        t0 = cols[pl.ds(4 * B_W + g * 16, 16)]
        t1 = cols[pl.ds(5 * B_W + g * 16, 16)]
        t2 = cols[pl.ds(6 * B_W + g * 16, 16)]

        n2 = v0 * v0 + v1 * v1 + v2 * v2
        y = _rsqrt16(n2)
        inv = 1.0 / (n2 * y + EPS)
        u0 = v0 * inv
        u1 = v1 * inv
        u2 = v2 * inv

        s, cc = _sincos16(a)
        oc = 1.0 - cc
        m = u0 * u0 + u1 * u1 + u2 * u2

        vals = (
            1.0 + oc * (u0 * u0 - m),
            oc * u0 * u1 - s * u2,
            oc * u0 * u2 + s * u1,
            t0,
            oc * u1 * u0 + s * u2,
            1.0 + oc * (u1 * u1 - m),
            oc * u1 * u2 - s * u0,
            t1,
            oc * u2 * u0 - s * u1,
            oc * u2 * u1 + s * u0,
            1.0 + oc * (u2 * u2 - m),
            t2,
            zero, zero, zero, one,
        )
        for f in range(16):
            stage[pl.ds(f * B_W + g * 16, 16)] = vals[f]

    for f in range(16):
        pltpu.sync_copy(stage.at[pl.ds(f * B_W, B_W)],
                        out_hbm.at[pl.ds(f * BATCH + base, B_W)])


@functools.lru_cache(maxsize=1)
def _sc_fused():
    return functools.partial(
        pl.kernel,
        out_type=jax.ShapeDtypeStruct((BATCH * 16,), jnp.float32),
        mesh=plsc.VectorSubcoreMesh(core_axis_name="c", subcore_axis_name="s"),
        scratch_types=[
            pltpu.VMEM((B_W,), jnp.int32),
            pltpu.VMEM((7 * B_W,), jnp.int32),
            pltpu.VMEM((7 * B_W,), jnp.float32),
            pltpu.VMEM((16 * B_W,), jnp.float32),
            pltpu.SemaphoreType.DMA,
        ],
    )(_sc_body)


def kernel(image_idx, W_v, W_alpha, W_T):
    idx = image_idx.astype(jnp.int32)
    pad = ((0, NPAD - N_TAB), (0, 0))
    Wvp = jnp.pad(W_v, pad)
    Wap = jnp.pad(W_alpha, pad)
    WTp = jnp.pad(W_T, pad)
    Z = jnp.concatenate([Wvp, Wap, WTp, Wap], axis=1)
    Y = Z.T.reshape(8, NPAD // CHUNK, CHUNK).transpose(1, 0, 2).reshape(-1)
    out = _sc_fused()(idx, Y)
    return out.reshape(16, BATCH).T.reshape(BATCH, 4, 4)


# speedup vs baseline: 109.5155x; 1.1186x over previous
import functools

import jax
import jax.numpy as jnp
from jax import lax
from jax.experimental import pallas as pl
from jax.experimental.pallas import tpu as pltpu
from jax.experimental.pallas import tpu_sc as plsc

EPS = 1e-08
BATCH = 16384
NC = 2
NS = 16
NW = NC * NS
B_W = BATCH // NW
CHUNK = 128
N_CHUNK = B_W // CHUNK
N_GROUP = B_W // 16

N_TAB = 1000000
NPAD = 1000064

_INV_PI = 0.31830988618379067
_PI_HI = 3.140625
_PI_LO = 9.67653589793e-4
_MAGIC = 0x5F3759DF


def _rsqrt16(x):
    i = lax.bitcast_convert_type(x, jnp.int32)
    i = jnp.full((16,), _MAGIC, jnp.int32) - lax.shift_right_arithmetic(i, 1)
    y = lax.bitcast_convert_type(i, jnp.float32)
    for _ in range(3):
        y = y * (1.5 - 0.5 * x * y * y)
    return y


def _sincos16(a):
    shift = jnp.where(a >= 0.0, 0.5, -0.5)
    n = (a * _INV_PI + shift).astype(jnp.int32)
    nf = n.astype(jnp.float32)
    r = (a - nf * _PI_HI) - nf * _PI_LO
    r2 = r * r
    s = r * (1.0 + r2 * (-1.0 / 6.0 + r2 * (1.0 / 120.0 + r2 * (-1.0 / 5040.0 + r2 * (1.0 / 362880.0)))))
    c = 1.0 + r2 * (-0.5 + r2 * (1.0 / 24.0 + r2 * (-1.0 / 720.0 + r2 * (1.0 / 40320.0 + r2 * (-1.0 / 3628800.0)))))
    sgn = jnp.where((n & 1) == 0, 1.0, -1.0)
    return s * sgn, c * sgn


def _sc_body(idx_hbm, y_hbm, out_hbm, idx_v, off_v, cols, stage, sem):
    wid = lax.axis_index("s") * NC + lax.axis_index("c")
    base = wid * B_W

    pltpu.sync_copy(idx_hbm.at[pl.ds(base, B_W)], idx_v)

    for g in range(N_GROUP):
        x = idx_v[pl.ds(g * 16, 16)]
        b = lax.shift_left(lax.shift_right_logical(x, 7), 10) + (x & 127)
        for f in range(7):
            off_v[pl.ds(f * B_W + g * 16, 16)] = b + (f * CHUNK)

    descs = []
    for c in range(N_CHUNK):
        for f in range(7):
            off_c = off_v.at[pl.ds(f * B_W + c * CHUNK, CHUNK)]
            descs.append(pltpu.async_copy(
                y_hbm.at[off_c],
                cols.at[pl.ds(f * B_W + c * CHUNK, CHUNK)], sem))
    for d in descs:
        d.wait()

    zero = jnp.zeros((16,), jnp.float32)
    one = jnp.full((16,), 1.0, jnp.float32)
    for g in range(N_GROUP):
        v0 = cols[pl.ds(0 * B_W + g * 16, 16)]
        v1 = cols[pl.ds(1 * B_W + g * 16, 16)]
        v2 = cols[pl.ds(2 * B_W + g * 16, 16)]
        a = cols[pl.ds(3 * B_W + g * 16, 16)]
        t0 = cols[pl.ds(4 * B_W + g * 16, 16)]
        t1 = cols[pl.ds(5 * B_W + g * 16, 16)]
        t2 = cols[pl.ds(6 * B_W + g * 16, 16)]

        n2 = v0 * v0 + v1 * v1 + v2 * v2
        y = _rsqrt16(n2)
        inv = 1.0 / (n2 * y + EPS)
        u0 = v0 * inv
        u1 = v1 * inv
        u2 = v2 * inv

        s, cc = _sincos16(a)
        oc = 1.0 - cc
        m = u0 * u0 + u1 * u1 + u2 * u2

        vals = (
            1.0 + oc * (u0 * u0 - m),
            oc * u0 * u1 - s * u2,
            oc * u0 * u2 + s * u1,
            t0,
            oc * u1 * u0 + s * u2,
            1.0 + oc * (u1 * u1 - m),
            oc * u1 * u2 - s * u0,
            t1,
            oc * u2 * u0 - s * u1,
            oc * u2 * u1 + s * u0,
            1.0 + oc * (u2 * u2 - m),
            t2,
            zero, zero, zero, one,
        )
        for f in range(16):
            stage[pl.ds(f * B_W + g * 16, 16)] = vals[f]

    for f in range(16):
        pltpu.sync_copy(stage.at[pl.ds(f * B_W, B_W)],
                        out_hbm.at[pl.ds(f * BATCH + base, B_W)])


@functools.lru_cache(maxsize=1)
def _sc_fused():
    return functools.partial(
        pl.kernel,
        out_type=jax.ShapeDtypeStruct((BATCH * 16,), jnp.float32),
        mesh=plsc.VectorSubcoreMesh(core_axis_name="c", subcore_axis_name="s"),
        scratch_types=[
            pltpu.VMEM((B_W,), jnp.int32),
            pltpu.VMEM((7 * B_W,), jnp.int32),
            pltpu.VMEM((7 * B_W,), jnp.float32),
            pltpu.VMEM((16 * B_W,), jnp.float32),
            pltpu.SemaphoreType.DMA,
        ],
    )(_sc_body)


def kernel(image_idx, W_v, W_alpha, W_T):
    idx = image_idx.astype(jnp.int32)
    pad = ((0, NPAD - N_TAB), (0, 0))
    Wvp = jnp.pad(W_v, pad)
    Wap = jnp.pad(W_alpha, pad)
    WTp = jnp.pad(W_T, pad)
    zcol = jnp.zeros((NPAD, 1), jnp.float32)
    Z = jnp.concatenate([Wvp, Wap, WTp, zcol], axis=1)
    Y = Z.T.reshape(8, NPAD // CHUNK, CHUNK).transpose(1, 0, 2).reshape(-1)
    out = _sc_fused()(idx, Y)
    return out.reshape(16, BATCH).T.reshape(BATCH, 4, 4)
